# baseline (speedup 1.0000x reference)
import functools

import jax
import jax.numpy as jnp
from jax import lax
from jax.experimental import pallas as pl
from jax.experimental.pallas import tpu as pltpu
from jax.experimental.pallas import tpu_sc as plsc

_NUM_WORKERS = 32
_MAIN = 24
_TAIL = _NUM_WORKERS * _MAIN


def kernel(indices, embedding_weight):
    batch, tokens = indices.shape
    vocab, dim = embedding_weight.shape
    rows = batch * tokens
    subdim = dim // 8

    idx_flat = indices.reshape(-1).astype(jnp.int32)
    mesh = plsc.VectorSubcoreMesh(core_axis_name="c", subcore_axis_name="s")

    @functools.partial(
        pl.kernel,
        mesh=mesh,
        out_type=jax.ShapeDtypeStruct((rows, dim), jnp.float32),
        scratch_types=[
            pltpu.VMEM((32,), jnp.int32),
            pltpu.VMEM((8, dim), jnp.float32),
            pltpu.VMEM((16, dim), jnp.float32),
            pltpu.VMEM_SHARED((16, dim), jnp.float32),
            pltpu.SemaphoreType.DMA,
            pltpu.SemaphoreType.DMA,
        ],
    )
    def gather_kernel(
        table_hbm, idx_hbm, out_hbm, idx_v, buf_a, buf_b, tail_sp, gsem, wsem
    ):
        c = lax.axis_index("c")
        s = lax.axis_index("s")
        wid = s * 2 + c
        off = wid * _MAIN
        fetches_tail = s < 2
        tail_blk = _TAIL + (s * 2 + c) * 8

        st0 = pltpu.async_copy(
            idx_hbm.at[pl.ds(off, _MAIN)], idx_v.at[pl.ds(0, _MAIN)], gsem
        )
        st1 = pltpu.async_copy(
            idx_hbm.at[pl.ds(jnp.where(fetches_tail, tail_blk, 0), 8)],
            idx_v.at[pl.ds(_MAIN, 8)],
            gsem,
        )
        st0.wait()
        st1.wait()

        g0 = pltpu.async_copy(
            table_hbm.at[idx_v.at[pl.ds(0, 8)]], buf_a, gsem
        )
        g1 = pltpu.async_copy(
            table_hbm.at[idx_v.at[pl.ds(8, 16)]], buf_b, gsem
        )
        g0.wait()
        w0 = pltpu.async_copy(buf_a, out_hbm.at[pl.ds(off, 8)], wsem)
        g1.wait()
        w1 = pltpu.async_copy(buf_b, out_hbm.at[pl.ds(off + 8, 16)], wsem)

        w0.wait()

        @pl.when(fetches_tail)
        def _():
            pltpu.async_copy(
                table_hbm.at[idx_v.at[pl.ds(_MAIN, 8)]], buf_a, gsem
            ).wait()
            pltpu.sync_copy(buf_a, tail_sp.at[pl.ds(s * 8, 8)])

        plsc.subcore_barrier()

        lblk = s // 8
        pcol = (s % 8) * subdim
        w2 = pltpu.async_copy(
            tail_sp.at[pl.ds(lblk * 8, 8), pl.ds(pcol, subdim)],
            out_hbm.at[
                pl.ds(_TAIL + (lblk * 2 + c) * 8, 8), pl.ds(pcol, subdim)
            ],
            wsem,
        )
        w1.wait()
        w2.wait()

    out = gather_kernel(embedding_weight, idx_flat)
    return out.reshape(batch, tokens, dim)


# speedup vs baseline: 1.0418x; 1.0418x over previous
import functools

import jax
import jax.numpy as jnp
from jax import lax
from jax.experimental import pallas as pl
from jax.experimental.pallas import tpu as pltpu
from jax.experimental.pallas import tpu_sc as plsc

_NUM_WORKERS = 32
_LIGHT = 28


def kernel(indices, embedding_weight):
    batch, tokens = indices.shape
    vocab, dim = embedding_weight.shape
    rows = batch * tokens
    assert _LIGHT * 24 + (_NUM_WORKERS - _LIGHT) * 32 == rows

    idx_flat = indices.reshape(-1).astype(jnp.int32)
    mesh = plsc.VectorSubcoreMesh(core_axis_name="c", subcore_axis_name="s")

    @functools.partial(
        pl.kernel,
        mesh=mesh,
        out_type=jax.ShapeDtypeStruct((rows, dim), jnp.float32),
        scratch_types=[
            pltpu.VMEM((32,), jnp.int32),
            pltpu.VMEM((8, dim), jnp.float32),
            pltpu.VMEM((16, dim), jnp.float32),
            pltpu.SemaphoreType.DMA,
            pltpu.SemaphoreType.DMA,
        ],
    )
    def gather_kernel(table_hbm, idx_hbm, out_hbm, idx_v, buf_a, buf_b, gsem, wsem):
        wid = lax.axis_index("s") * 2 + lax.axis_index("c")
        heavy = wid >= _LIGHT
        off = jnp.where(heavy, _LIGHT * 24 + (wid - _LIGHT) * 32, wid * 24)

        pltpu.sync_copy(idx_hbm.at[pl.ds(off, 32)], idx_v)

        def gather(i0, n, buf):
            return pltpu.async_copy(
                table_hbm.at[idx_v.at[pl.ds(i0, n)]], buf, gsem
            )

        g0 = gather(0, 8, buf_a)
        g1 = gather(8, 16, buf_b)
        g0.wait()
        w0 = pltpu.async_copy(buf_a, out_hbm.at[pl.ds(off, 8)], wsem)
        g1.wait()
        w1 = pltpu.async_copy(buf_b, out_hbm.at[pl.ds(off + 8, 16)], wsem)
        w0.wait()

        @pl.when(heavy)
        def _():
            g2 = gather(24, 8, buf_a)
            g2.wait()
            pltpu.async_copy(
                buf_a, out_hbm.at[pl.ds(off + 24, 8)], wsem
            ).wait()

        w1.wait()

    out = gather_kernel(embedding_weight, idx_flat)
    return out.reshape(batch, tokens, dim)
